# Initial kernel scaffold; baseline (speedup 1.0000x reference)
import jax
import jax.numpy as jnp
from jax.experimental import pallas as pl


def kernel(x, edge_index, edge_attr, W1, b1, W2, b2, We, be, Wo, bo, Wf, bf):
    raise NotImplementedError("write your pallas kernel here")


# speedup vs baseline: 11.0511x; 11.0511x over previous
import functools

import jax
import jax.numpy as jnp
from jax import lax
from jax.experimental import pallas as pl
from jax.experimental.pallas import tpu as pltpu
from jax.experimental.pallas import tpu_sc as plsc

N = 10000
E = 320000
D_NODE = 128

NPAD = 10240
NT = 32
BLKL = 1024
NOP = 10
EPT = BLKL * NOP
EPAD = NT * EPT
NSLICE = NPAD // 16
RB = 4
LA = 2

_mesh = plsc.VectorSubcoreMesh(core_axis_name="c", subcore_axis_name="s")
_sc_params = pltpu.CompilerParams(use_tc_tiling_on_sc=False)


def _wid(cid, sid):
    return sid * 2 + cid


@functools.partial(
    pl.kernel,
    out_type=jax.ShapeDtypeStruct((2, NPAD, 8), jnp.float32),
    mesh=_mesh,
    compiler_params=_sc_params,
    scratch_types=[
        pltpu.VMEM((NOP, BLKL), jnp.int32),
        pltpu.VMEM((BLKL, 8), jnp.float32),
        pltpu.VMEM_SHARED((NPAD, 8), jnp.float32),
        pltpu.SemaphoreType.DMA,
    ],
)
def _sc_degree(dst2d, ones2, zeros8, out, idx_v, ones_v, acc, sem):
    cid = lax.axis_index("c")
    sid = lax.axis_index("s")
    w = _wid(cid, sid)
    pltpu.sync_copy(zeros8, acc.at[pl.ds(sid * NSLICE, NSLICE)])
    pltpu.sync_copy(dst2d.at[pl.ds(w * NOP, NOP)], idx_v)
    pltpu.sync_copy(ones2, ones_v)
    plsc.subcore_barrier()
    descs = []
    for k in range(NOP):
        descs.append(pltpu.async_copy(
            ones_v, acc.at[idx_v.at[k]], sem, add=True))
    for d in descs:
        d.wait()
    plsc.subcore_barrier()
    pltpu.sync_copy(acc.at[pl.ds(sid * NSLICE, NSLICE)],
                    out.at[cid, pl.ds(sid * NSLICE, NSLICE)])


def _make_sc_agg(width):
    @functools.partial(
        pl.kernel,
        out_type=jax.ShapeDtypeStruct((2, NPAD, width), jnp.float32),
        mesh=_mesh,
        compiler_params=_sc_params,
        scratch_types=[
            pltpu.VMEM((NOP, BLKL), jnp.int32),
            pltpu.VMEM((NOP, BLKL), jnp.int32),
            [pltpu.VMEM((BLKL, width), jnp.float32)] * RB,
            [pltpu.SemaphoreType.DMA] * RB,
            [pltpu.SemaphoreType.DMA] * RB,
            pltpu.VMEM_SHARED((NPAD, width), jnp.float32),
        ],
    )
    def _sc_agg(table, src2d, dst2d, zeros, out,
                src_v, dst_v, bufs, gsems, wsems, acc):
        cid = lax.axis_index("c")
        sid = lax.axis_index("s")
        w = _wid(cid, sid)
        pltpu.sync_copy(zeros, acc.at[pl.ds(sid * NSLICE, NSLICE)])
        pltpu.sync_copy(src2d.at[pl.ds(w * NOP, NOP)], src_v)
        pltpu.sync_copy(dst2d.at[pl.ds(w * NOP, NOP)], dst_v)
        plsc.subcore_barrier()

        def gather(k):
            r = k % RB
            return pltpu.async_copy(table.at[src_v.at[k]], bufs[r], gsems[r])

        gd = {k: gather(k) for k in range(LA)}
        sd = {}
        drained = set()
        for k in range(NOP):
            r = k % RB
            gd[k].wait()
            sd[k] = pltpu.async_copy(
                bufs[r], acc.at[dst_v.at[k]], wsems[r], add=True)
            kn = k + LA
            if kn < NOP:
                prev = kn - RB
                if prev >= 0:
                    sd[prev].wait()
                    drained.add(prev)
                gd[kn] = gather(kn)
        for k in range(NOP):
            if k not in drained:
                sd[k].wait()
        plsc.subcore_barrier()
        pltpu.sync_copy(acc.at[pl.ds(sid * NSLICE, NSLICE)],
                        out.at[cid, pl.ds(sid * NSLICE, NSLICE)])

    return _sc_agg


_sc_agg16 = _make_sc_agg(16)
_sc_agg8 = _make_sc_agg(8)


@functools.partial(
    pl.kernel,
    out_type=(jax.ShapeDtypeStruct((NT * NOP, BLKL, 8), jnp.float32),
              jax.ShapeDtypeStruct((NT * NOP, BLKL, 8), jnp.float32)),
    mesh=_mesh,
    compiler_params=_sc_params,
    scratch_types=[
        pltpu.VMEM((NOP, BLKL), jnp.int32),
        pltpu.VMEM((NOP, BLKL), jnp.int32),
        [pltpu.VMEM((BLKL, 8), jnp.float32)] * RB,
        [pltpu.VMEM((BLKL, 8), jnp.float32)] * RB,
        [pltpu.SemaphoreType.DMA] * RB,
        [pltpu.SemaphoreType.DMA] * RB,
        [pltpu.SemaphoreType.DMA] * RB,
        [pltpu.SemaphoreType.DMA] * RB,
    ],
)
def _sc_gather(table, src2d, dst2d, out_s, out_d,
               src_v, dst_v, bufs_s, bufs_d, gsems_s, gsems_d,
               wsems_s, wsems_d):
    cid = lax.axis_index("c")
    sid = lax.axis_index("s")
    w = _wid(cid, sid)
    pltpu.sync_copy(src2d.at[pl.ds(w * NOP, NOP)], src_v)
    pltpu.sync_copy(dst2d.at[pl.ds(w * NOP, NOP)], dst_v)

    def gather(k, idx_v, bufs, gsems):
        r = k % RB
        return pltpu.async_copy(table.at[idx_v.at[k]], bufs[r], gsems[r])

    gds = {k: gather(k, src_v, bufs_s, gsems_s) for k in range(LA)}
    gdd = {k: gather(k, dst_v, bufs_d, gsems_d) for k in range(LA)}
    wds, wdd = {}, {}
    drained = set()
    for k in range(NOP):
        r = k % RB
        row = w * NOP + k
        gds[k].wait()
        wds[k] = pltpu.async_copy(bufs_s[r], out_s.at[row], wsems_s[r])
        gdd[k].wait()
        wdd[k] = pltpu.async_copy(bufs_d[r], out_d.at[row], wsems_d[r])
        kn = k + LA
        if kn < NOP:
            prev = kn - RB
            if prev >= 0:
                wds[prev].wait()
                wdd[prev].wait()
                drained.add(prev)
            gds[kn] = gather(kn, src_v, bufs_s, gsems_s)
            gdd[kn] = gather(kn, dst_v, bufs_d, gsems_d)
    for k in range(NOP):
        if k not in drained:
            wds[k].wait()
            wdd[k].wait()


def _tc_emlp_body(ea_ref, we_ref, be_ref, out_ref):
    out_ref[...] = jnp.maximum(
        jnp.dot(ea_ref[...], we_ref[...], preferred_element_type=jnp.float32)
        + be_ref[...], 0.0)


def _tc1_body(deg_ref, x_ref, w1_ref, xws_ref, dinv_ref):
    dp = deg_ref[...]
    deg = dp[0] + dp[1] + 1.0
    dinv8 = lax.rsqrt(deg)
    xw = jnp.dot(x_ref[...], w1_ref[...], preferred_element_type=jnp.float32)
    dinv16 = jnp.concatenate([dinv8, dinv8], axis=1)
    xws_ref[...] = xw * dinv16
    dinv_ref[...] = dinv8


def _tc2_body(agg_ref, xws_ref, dinv_ref, w2_ref, b1_ref, out_ref):
    ap = agg_ref[...]
    dinv8 = dinv_ref[...]
    dinv16 = jnp.concatenate([dinv8, dinv8], axis=1)
    conv = (ap[0] + ap[1] + xws_ref[...]) * dinv16 + b1_ref[...]
    h1 = jnp.maximum(conv, 0.0)
    out_ref[...] = jnp.dot(h1, w2_ref[...],
                           preferred_element_type=jnp.float32) * dinv8


def _tc3_body(agg_ref, h1ws_ref, dinv_ref, b2_ref, out_ref):
    ap = agg_ref[...]
    conv = (ap[0] + ap[1] + h1ws_ref[...]) * dinv_ref[...] + b2_ref[...]
    out_ref[...] = jnp.maximum(conv, 0.0)


def _tc4_body(e_ref, hs_ref, hd_ref, wo_ref, bo_ref, wf_ref, bf_ref, out_ref):
    wo = wo_ref[...]
    f = (jnp.dot(hs_ref[...], wo[0:8], preferred_element_type=jnp.float32)
         + jnp.dot(hd_ref[...], wo[8:16], preferred_element_type=jnp.float32)
         + jnp.dot(e_ref[...], wo[16:24], preferred_element_type=jnp.float32)
         + bo_ref[...])
    f = jnp.maximum(f, 0.0)
    out_ref[...] = jnp.dot(f, wf_ref[...],
                           preferred_element_type=jnp.float32) + bf_ref[...]


_EBLK = 8000


def kernel(x, edge_index, edge_attr, W1, b1, W2, b2, We, be, Wo, bo, Wf, bf):
    src = edge_index[0]
    dst = edge_index[1]
    pad = jnp.full((EPAD - E,), N, jnp.int32)
    src2d = jnp.concatenate([src, pad]).reshape(NT * NOP, BLKL)
    dst2d = jnp.concatenate([dst, pad]).reshape(NT * NOP, BLKL)
    x_pad = jnp.pad(x, ((0, NPAD - N), (0, 0)))
    ones2 = jnp.ones((BLKL, 8), jnp.float32)
    z8 = jnp.zeros((NSLICE, 8), jnp.float32)
    z16 = jnp.zeros((NSLICE, 16), jnp.float32)

    nblk = E // _EBLK
    e16 = pl.pallas_call(
        _tc_emlp_body,
        grid=(nblk,),
        in_specs=[
            pl.BlockSpec((_EBLK, 16), lambda i: (i, 0)),
            pl.BlockSpec((16, 8), lambda i: (0, 0)),
            pl.BlockSpec((1, 8), lambda i: (0, 0)),
        ],
        out_specs=pl.BlockSpec((_EBLK, 8), lambda i: (i, 0)),
        out_shape=jax.ShapeDtypeStruct((E, 8), jnp.float32),
    )(edge_attr, We, be.reshape(1, 8))

    deg_p = _sc_degree(dst2d, ones2, z8)

    xws, dinv8 = pl.pallas_call(
        _tc1_body,
        out_shape=(jax.ShapeDtypeStruct((NPAD, 16), jnp.float32),
                   jax.ShapeDtypeStruct((NPAD, 8), jnp.float32)),
    )(deg_p, x_pad, W1)

    agg1_p = _sc_agg16(xws, src2d, dst2d, z16)

    h1ws = pl.pallas_call(
        _tc2_body,
        out_shape=jax.ShapeDtypeStruct((NPAD, 8), jnp.float32),
    )(agg1_p, xws, dinv8, W2, b1.reshape(1, 16))

    agg2_p = _sc_agg8(h1ws, src2d, dst2d, z8)

    h2 = pl.pallas_call(
        _tc3_body,
        out_shape=jax.ShapeDtypeStruct((NPAD, 8), jnp.float32),
    )(agg2_p, h1ws, dinv8, b2.reshape(1, 8))

    h2s3, h2d3 = _sc_gather(h2, src2d, dst2d)
    h2s = h2s3.reshape(EPAD, 8)
    h2d = h2d3.reshape(EPAD, 8)

    out = pl.pallas_call(
        _tc4_body,
        grid=(nblk,),
        in_specs=[
            pl.BlockSpec((_EBLK, 8), lambda i: (i, 0)),
            pl.BlockSpec((_EBLK, 8), lambda i: (i, 0)),
            pl.BlockSpec((_EBLK, 8), lambda i: (i, 0)),
            pl.BlockSpec((24, 8), lambda i: (0, 0)),
            pl.BlockSpec((1, 8), lambda i: (0, 0)),
            pl.BlockSpec((8, 1), lambda i: (0, 0)),
            pl.BlockSpec((1, 1), lambda i: (0, 0)),
        ],
        out_specs=pl.BlockSpec((_EBLK, 1), lambda i: (i, 0)),
        out_shape=jax.ShapeDtypeStruct((E, 1), jnp.float32),
    )(e16, h2s, h2d, Wo, bo.reshape(1, 8), Wf, bf.reshape(1, 1))

    return out


# speedup vs baseline: 22.6422x; 2.0489x over previous
import functools

import jax
import jax.numpy as jnp
from jax import lax
from jax.experimental import pallas as pl
from jax.experimental.pallas import tpu as pltpu
from jax.experimental.pallas import tpu_sc as plsc

N = 10000
E = 320000
D_NODE = 128

NPAD = 10240
NT = 32
BLKL = 1024
NOP = 10
EPT = BLKL * NOP
EPAD = NT * EPT
NSLICE = NPAD // 16
RB = 4
LA = 2

_mesh = plsc.VectorSubcoreMesh(core_axis_name="c", subcore_axis_name="s")
_sc_params = pltpu.CompilerParams(use_tc_tiling_on_sc=False)


def _wid(cid, sid):
    return sid * 2 + cid


@functools.partial(
    pl.kernel,
    out_type=jax.ShapeDtypeStruct((2, NPAD, 8), jnp.float32),
    mesh=_mesh,
    compiler_params=_sc_params,
    scratch_types=[
        pltpu.VMEM((NOP, BLKL), jnp.int32),
        pltpu.VMEM((BLKL, 8), jnp.float32),
        pltpu.VMEM_SHARED((NPAD, 8), jnp.float32),
        pltpu.SemaphoreType.DMA,
    ],
)
def _sc_degree(dst2d, ones2, zeros8, out, idx_v, ones_v, acc, sem):
    cid = lax.axis_index("c")
    sid = lax.axis_index("s")
    w = _wid(cid, sid)
    pltpu.sync_copy(zeros8, acc.at[pl.ds(sid * NSLICE, NSLICE)])
    pltpu.sync_copy(dst2d.at[pl.ds(w * NOP, NOP)], idx_v)
    pltpu.sync_copy(ones2, ones_v)
    plsc.subcore_barrier()
    descs = []
    for k in range(NOP):
        descs.append(pltpu.async_copy(
            ones_v, acc.at[idx_v.at[k]], sem, add=True))
    for d in descs:
        d.wait()
    plsc.subcore_barrier()
    pltpu.sync_copy(acc.at[pl.ds(sid * NSLICE, NSLICE)],
                    out.at[cid, pl.ds(sid * NSLICE, NSLICE)])


def _make_sc_agg(width):
    @functools.partial(
        pl.kernel,
        out_type=jax.ShapeDtypeStruct((2, NPAD, width), jnp.float32),
        mesh=_mesh,
        compiler_params=_sc_params,
        scratch_types=[
            pltpu.VMEM((NOP, BLKL), jnp.int32),
            pltpu.VMEM((NOP, BLKL), jnp.int32),
            [pltpu.VMEM((BLKL, width), jnp.float32)] * RB,
            [pltpu.SemaphoreType.DMA] * RB,
            [pltpu.SemaphoreType.DMA] * RB,
            pltpu.VMEM_SHARED((NPAD, width), jnp.float32),
        ],
    )
    def _sc_agg(table, src2d, dst2d, zeros, out,
                src_v, dst_v, bufs, gsems, wsems, acc):
        cid = lax.axis_index("c")
        sid = lax.axis_index("s")
        w = _wid(cid, sid)
        pltpu.sync_copy(zeros, acc.at[pl.ds(sid * NSLICE, NSLICE)])
        pltpu.sync_copy(src2d.at[pl.ds(w * NOP, NOP)], src_v)
        pltpu.sync_copy(dst2d.at[pl.ds(w * NOP, NOP)], dst_v)
        plsc.subcore_barrier()

        def gather(k):
            r = k % RB
            return pltpu.async_copy(table.at[src_v.at[k]], bufs[r], gsems[r])

        gd = {k: gather(k) for k in range(LA)}
        sd = {}
        drained = set()
        for k in range(NOP):
            r = k % RB
            gd[k].wait()
            sd[k] = pltpu.async_copy(
                bufs[r], acc.at[dst_v.at[k]], wsems[r], add=True)
            kn = k + LA
            if kn < NOP:
                prev = kn - RB
                if prev >= 0:
                    sd[prev].wait()
                    drained.add(prev)
                gd[kn] = gather(kn)
        for k in range(NOP):
            if k not in drained:
                sd[k].wait()
        plsc.subcore_barrier()
        pltpu.sync_copy(acc.at[pl.ds(sid * NSLICE, NSLICE)],
                        out.at[cid, pl.ds(sid * NSLICE, NSLICE)])

    return _sc_agg


_sc_agg16 = _make_sc_agg(16)
_sc_agg8 = _make_sc_agg(8)


@functools.partial(
    pl.kernel,
    out_type=(jax.ShapeDtypeStruct((NT * NOP, BLKL, 8), jnp.float32),
              jax.ShapeDtypeStruct((NT * NOP, BLKL, 8), jnp.float32)),
    mesh=_mesh,
    compiler_params=_sc_params,
    scratch_types=[
        pltpu.VMEM((NOP, BLKL), jnp.int32),
        pltpu.VMEM((NOP, BLKL), jnp.int32),
        [pltpu.VMEM((BLKL, 8), jnp.float32)] * RB,
        [pltpu.VMEM((BLKL, 8), jnp.float32)] * RB,
        [pltpu.SemaphoreType.DMA] * RB,
        [pltpu.SemaphoreType.DMA] * RB,
        [pltpu.SemaphoreType.DMA] * RB,
        [pltpu.SemaphoreType.DMA] * RB,
    ],
)
def _sc_gather(table, src2d, dst2d, out_s, out_d,
               src_v, dst_v, bufs_s, bufs_d, gsems_s, gsems_d,
               wsems_s, wsems_d):
    cid = lax.axis_index("c")
    sid = lax.axis_index("s")
    w = _wid(cid, sid)
    pltpu.sync_copy(src2d.at[pl.ds(w * NOP, NOP)], src_v)
    pltpu.sync_copy(dst2d.at[pl.ds(w * NOP, NOP)], dst_v)

    def gather(k, idx_v, bufs, gsems):
        r = k % RB
        return pltpu.async_copy(table.at[idx_v.at[k]], bufs[r], gsems[r])

    gds = {k: gather(k, src_v, bufs_s, gsems_s) for k in range(LA)}
    gdd = {k: gather(k, dst_v, bufs_d, gsems_d) for k in range(LA)}
    wds, wdd = {}, {}
    drained = set()
    for k in range(NOP):
        r = k % RB
        row = w * NOP + k
        gds[k].wait()
        wds[k] = pltpu.async_copy(bufs_s[r], out_s.at[row], wsems_s[r])
        gdd[k].wait()
        wdd[k] = pltpu.async_copy(bufs_d[r], out_d.at[row], wsems_d[r])
        kn = k + LA
        if kn < NOP:
            prev = kn - RB
            if prev >= 0:
                wds[prev].wait()
                wdd[prev].wait()
                drained.add(prev)
            gds[kn] = gather(kn, src_v, bufs_s, gsems_s)
            gdd[kn] = gather(kn, dst_v, bufs_d, gsems_d)
    for k in range(NOP):
        if k not in drained:
            wds[k].wait()
            wdd[k].wait()


def _tc1_body(deg_ref, x_ref, w1_ref, xws_ref, dinv_ref):
    dp = deg_ref[...]
    deg = dp[0] + dp[1] + 1.0
    dinv8 = lax.rsqrt(deg)
    xw = jnp.dot(x_ref[...], w1_ref[...], preferred_element_type=jnp.float32)
    dinv16 = jnp.concatenate([dinv8, dinv8], axis=1)
    xws_ref[...] = xw * dinv16
    dinv_ref[...] = dinv8


def _tc2_body(agg_ref, xws_ref, dinv_ref, w2_ref, b1_ref, out_ref):
    ap = agg_ref[...]
    dinv8 = dinv_ref[...]
    dinv16 = jnp.concatenate([dinv8, dinv8], axis=1)
    conv = (ap[0] + ap[1] + xws_ref[...]) * dinv16 + b1_ref[...]
    h1 = jnp.maximum(conv, 0.0)
    out_ref[...] = jnp.dot(h1, w2_ref[...],
                           preferred_element_type=jnp.float32) * dinv8


def _tc3_body(agg_ref, h1ws_ref, dinv_ref, b2_ref, out_ref):
    ap = agg_ref[...]
    conv = (ap[0] + ap[1] + h1ws_ref[...]) * dinv_ref[...] + b2_ref[...]
    out_ref[...] = jnp.maximum(conv, 0.0)


def _tc4_body(ea_ref, hs_ref, hd_ref, kwe_ref, be_ref, kwo3_ref, bo_ref,
              kwo1_ref, kwo2_ref, kwf_ref, bf_ref, out_ref):
    e8 = jnp.maximum(
        jnp.dot(ea_ref[...], kwe_ref[...], preferred_element_type=jnp.float32)
        + be_ref[...], 0.0)
    q = jnp.dot(e8, kwo3_ref[...],
                preferred_element_type=jnp.float32) + bo_ref[...]
    f = (jnp.dot(hs_ref[...], kwo1_ref[...],
                 preferred_element_type=jnp.float32)
         + jnp.dot(hd_ref[...], kwo2_ref[...],
                   preferred_element_type=jnp.float32)
         + q)
    f = jnp.maximum(f, 0.0)
    out_ref[...] = jnp.dot(f, kwf_ref[...],
                           preferred_element_type=jnp.float32) + bf_ref[...]


_EBLK = 6400


def kernel(x, edge_index, edge_attr, W1, b1, W2, b2, We, be, Wo, bo, Wf, bf):
    src = edge_index[0]
    dst = edge_index[1]
    pad = jnp.full((EPAD - E,), N, jnp.int32)
    src2d = jnp.concatenate([src, pad]).reshape(NT * NOP, BLKL)
    dst2d = jnp.concatenate([dst, pad]).reshape(NT * NOP, BLKL)
    x_pad = jnp.pad(x, ((0, NPAD - N), (0, 0)))
    ones2 = jnp.ones((BLKL, 8), jnp.float32)
    z8 = jnp.zeros((NSLICE, 8), jnp.float32)
    z16 = jnp.zeros((NSLICE, 16), jnp.float32)

    nblk = E // _EBLK
    ea_p = edge_attr.reshape(E // 16, 256)

    deg_p = _sc_degree(dst2d, ones2, z8)

    xws, dinv8 = pl.pallas_call(
        _tc1_body,
        out_shape=(jax.ShapeDtypeStruct((NPAD, 16), jnp.float32),
                   jax.ShapeDtypeStruct((NPAD, 8), jnp.float32)),
    )(deg_p, x_pad, W1)

    agg1_p = _sc_agg16(xws, src2d, dst2d, z16)

    h1ws = pl.pallas_call(
        _tc2_body,
        out_shape=jax.ShapeDtypeStruct((NPAD, 8), jnp.float32),
    )(agg1_p, xws, dinv8, W2, b1.reshape(1, 16))

    agg2_p = _sc_agg8(h1ws, src2d, dst2d, z8)

    h2 = pl.pallas_call(
        _tc3_body,
        out_shape=jax.ShapeDtypeStruct((NPAD, 8), jnp.float32),
    )(agg2_p, h1ws, dinv8, b2.reshape(1, 8))

    h2s3, h2d3 = _sc_gather(h2, src2d, dst2d)
    h2s_p = h2s3.reshape(EPAD // 16, 128)
    h2d_p = h2d3.reshape(EPAD // 16, 128)

    eye16 = jnp.eye(16, dtype=jnp.float32)
    kwe = jnp.kron(eye16, We)
    kwo1 = jnp.kron(eye16, Wo[0:8])
    kwo2 = jnp.kron(eye16, Wo[8:16])
    kwo3 = jnp.kron(eye16, Wo[16:24])
    kwf = jnp.kron(eye16, Wf)
    be16 = jnp.tile(be, 16).reshape(1, 128)
    bo16 = jnp.tile(bo, 16).reshape(1, 128)
    bf16 = jnp.tile(bf, 16).reshape(1, 16)

    pblk = _EBLK // 16
    out_p = pl.pallas_call(
        _tc4_body,
        grid=(nblk,),
        in_specs=[
            pl.BlockSpec((pblk, 256), lambda i: (i, 0)),
            pl.BlockSpec((pblk, 128), lambda i: (i, 0)),
            pl.BlockSpec((pblk, 128), lambda i: (i, 0)),
            pl.BlockSpec((256, 128), lambda i: (0, 0)),
            pl.BlockSpec((1, 128), lambda i: (0, 0)),
            pl.BlockSpec((128, 128), lambda i: (0, 0)),
            pl.BlockSpec((1, 128), lambda i: (0, 0)),
            pl.BlockSpec((128, 128), lambda i: (0, 0)),
            pl.BlockSpec((128, 128), lambda i: (0, 0)),
            pl.BlockSpec((128, 16), lambda i: (0, 0)),
            pl.BlockSpec((1, 16), lambda i: (0, 0)),
        ],
        out_specs=pl.BlockSpec((pblk, 16), lambda i: (i, 0)),
        out_shape=jax.ShapeDtypeStruct((E // 16, 16), jnp.float32),
    )(ea_p, h2s_p, h2d_p, kwe, be16, kwo3, bo16, kwo1, kwo2, kwf, bf16)

    return out_p.reshape(E, 1)


# speedup vs baseline: 25.5173x; 1.1270x over previous
import functools

import jax
import jax.numpy as jnp
from jax import lax
from jax.experimental import pallas as pl
from jax.experimental.pallas import tpu as pltpu
from jax.experimental.pallas import tpu_sc as plsc

N = 10000
E = 320000
D_NODE = 128

NPAD = 10240
NT = 32
BLKL = 1024
NOP = 10
EPT = BLKL * NOP
EPAD = NT * EPT
NSLICE = NPAD // 16
RB = 4
LA = 2

_mesh = plsc.VectorSubcoreMesh(core_axis_name="c", subcore_axis_name="s")
_sc_params = pltpu.CompilerParams(use_tc_tiling_on_sc=False)


def _wid(cid, sid):
    return sid * 2 + cid


@functools.partial(
    pl.kernel,
    out_type=jax.ShapeDtypeStruct((2, NPAD, 16), jnp.float32),
    mesh=_mesh,
    compiler_params=_sc_params,
    scratch_types=[
        pltpu.VMEM((NOP, BLKL), jnp.int32),
        pltpu.VMEM((BLKL, 16), jnp.float32),
        pltpu.VMEM_SHARED((NPAD, 16), jnp.float32),
        pltpu.SemaphoreType.DMA,
    ],
)
def _sc_degree(dst2d, ones2, zeros16, out, idx_v, ones_v, acc, sem):
    cid = lax.axis_index("c")
    sid = lax.axis_index("s")
    w = _wid(cid, sid)
    pltpu.sync_copy(zeros16, acc.at[pl.ds(sid * NSLICE, NSLICE)])
    pltpu.sync_copy(dst2d.at[pl.ds(w * NOP, NOP)], idx_v)
    pltpu.sync_copy(ones2, ones_v)
    plsc.subcore_barrier()
    descs = []
    for k in range(NOP):
        descs.append(pltpu.async_copy(
            ones_v, acc.at[idx_v.at[k]], sem, add=True))
    for d in descs:
        d.wait()
    plsc.subcore_barrier()
    pltpu.sync_copy(acc.at[pl.ds(sid * NSLICE, NSLICE)],
                    out.at[cid, pl.ds(sid * NSLICE, NSLICE)])


def _make_sc_agg(width):
    @functools.partial(
        pl.kernel,
        out_type=jax.ShapeDtypeStruct((2, NPAD, width), jnp.float32),
        mesh=_mesh,
        compiler_params=_sc_params,
        scratch_types=[
            pltpu.VMEM((NOP, BLKL), jnp.int32),
            pltpu.VMEM((NOP, BLKL), jnp.int32),
            [pltpu.VMEM((BLKL, width), jnp.float32)] * RB,
            [pltpu.SemaphoreType.DMA] * RB,
            [pltpu.SemaphoreType.DMA] * RB,
            pltpu.VMEM_SHARED((NPAD, width), jnp.float32),
        ],
    )
    def _sc_agg(table, src2d, dst2d, zeros, out,
                src_v, dst_v, bufs, gsems, wsems, acc):
        cid = lax.axis_index("c")
        sid = lax.axis_index("s")
        w = _wid(cid, sid)
        pltpu.sync_copy(zeros, acc.at[pl.ds(sid * NSLICE, NSLICE)])
        pltpu.sync_copy(src2d.at[pl.ds(w * NOP, NOP)], src_v)
        pltpu.sync_copy(dst2d.at[pl.ds(w * NOP, NOP)], dst_v)
        plsc.subcore_barrier()

        def gather(k):
            r = k % RB
            return pltpu.async_copy(table.at[src_v.at[k]], bufs[r], gsems[r])

        gd = {k: gather(k) for k in range(LA)}
        sd = {}
        drained = set()
        for k in range(NOP):
            r = k % RB
            gd[k].wait()
            sd[k] = pltpu.async_copy(
                bufs[r], acc.at[dst_v.at[k]], wsems[r], add=True)
            kn = k + LA
            if kn < NOP:
                prev = kn - RB
                if prev >= 0:
                    sd[prev].wait()
                    drained.add(prev)
                gd[kn] = gather(kn)
        for k in range(NOP):
            if k not in drained:
                sd[k].wait()
        plsc.subcore_barrier()
        pltpu.sync_copy(acc.at[pl.ds(sid * NSLICE, NSLICE)],
                        out.at[cid, pl.ds(sid * NSLICE, NSLICE)])

    return _sc_agg


_sc_agg16 = _make_sc_agg(16)
_sc_agg8 = _make_sc_agg(8)


@functools.partial(
    pl.kernel,
    out_type=(jax.ShapeDtypeStruct((NT * NOP, BLKL, 8), jnp.float32),
              jax.ShapeDtypeStruct((NT * NOP, BLKL, 8), jnp.float32)),
    mesh=_mesh,
    compiler_params=_sc_params,
    scratch_types=[
        pltpu.VMEM((NOP, BLKL), jnp.int32),
        pltpu.VMEM((NOP, BLKL), jnp.int32),
        [pltpu.VMEM((BLKL, 8), jnp.float32)] * RB,
        [pltpu.VMEM((BLKL, 8), jnp.float32)] * RB,
        [pltpu.SemaphoreType.DMA] * RB,
        [pltpu.SemaphoreType.DMA] * RB,
        [pltpu.SemaphoreType.DMA] * RB,
        [pltpu.SemaphoreType.DMA] * RB,
    ],
)
def _sc_gather(table, src2d, dst2d, out_s, out_d,
               src_v, dst_v, bufs_s, bufs_d, gsems_s, gsems_d,
               wsems_s, wsems_d):
    cid = lax.axis_index("c")
    sid = lax.axis_index("s")
    w = _wid(cid, sid)
    pltpu.sync_copy(src2d.at[pl.ds(w * NOP, NOP)], src_v)
    pltpu.sync_copy(dst2d.at[pl.ds(w * NOP, NOP)], dst_v)

    def gather(k, idx_v, bufs, gsems):
        r = k % RB
        return pltpu.async_copy(table.at[idx_v.at[k]], bufs[r], gsems[r])

    gds = {k: gather(k, src_v, bufs_s, gsems_s) for k in range(LA)}
    gdd = {k: gather(k, dst_v, bufs_d, gsems_d) for k in range(LA)}
    wds, wdd = {}, {}
    drained = set()
    for k in range(NOP):
        r = k % RB
        row = w * NOP + k
        gds[k].wait()
        wds[k] = pltpu.async_copy(bufs_s[r], out_s.at[row], wsems_s[r])
        gdd[k].wait()
        wdd[k] = pltpu.async_copy(bufs_d[r], out_d.at[row], wsems_d[r])
        kn = k + LA
        if kn < NOP:
            prev = kn - RB
            if prev >= 0:
                wds[prev].wait()
                wdd[prev].wait()
                drained.add(prev)
            gds[kn] = gather(kn, src_v, bufs_s, gsems_s)
            gdd[kn] = gather(kn, dst_v, bufs_d, gsems_d)
    for k in range(NOP):
        if k not in drained:
            wds[k].wait()
            wdd[k].wait()


def _tc1_body(deg_ref, x_ref, kw1_ref, xws_ref, dinv_ref):
    dp = deg_ref[...]
    dinv16 = lax.rsqrt(dp[0] + dp[1] + 1.0)
    xw = jnp.dot(x_ref[...], kw1_ref[...],
                 preferred_element_type=jnp.float32)
    xws_ref[...] = xw * dinv16
    dinv_ref[...] = dinv16


def _tc2_body(agg_ref, xws_ref, dinv_ref, kw2_ref, t8_ref, b1_ref, out_ref):
    ap = agg_ref[...]
    dinv16 = dinv_ref[...]
    conv = (ap[0] + ap[1] + xws_ref[...]) * dinv16 + b1_ref[...]
    h1 = jnp.maximum(conv, 0.0)
    h1w = jnp.dot(h1, kw2_ref[...],
                  preferred_element_type=jnp.float32)
    dinv8 = jnp.dot(dinv16, t8_ref[...],
                    preferred_element_type=jnp.float32)
    out_ref[...] = h1w * dinv8


def _tc3_body(agg_ref, h1ws_ref, dinv_ref, t8_ref, b2_ref, out_ref):
    ap = agg_ref[...]
    dinv8 = jnp.dot(dinv_ref[...], t8_ref[...],
                    preferred_element_type=jnp.float32)
    conv = (ap[0] + ap[1] + h1ws_ref[...]) * dinv8 + b2_ref[...]
    out_ref[...] = jnp.maximum(conv, 0.0)


def _tc4_body(ea_ref, hs_ref, hd_ref, kwe_ref, be_ref, kwo3_ref, bo_ref,
              kwo1_ref, kwo2_ref, kwf_ref, bf_ref, out_ref):
    e8 = jnp.maximum(
        jnp.dot(ea_ref[...], kwe_ref[...], preferred_element_type=jnp.float32)
        + be_ref[...], 0.0)
    q = jnp.dot(e8, kwo3_ref[...],
                preferred_element_type=jnp.float32) + bo_ref[...]
    f = (jnp.dot(hs_ref[...], kwo1_ref[...],
                 preferred_element_type=jnp.float32)
         + jnp.dot(hd_ref[...], kwo2_ref[...],
                   preferred_element_type=jnp.float32)
         + q)
    f = jnp.maximum(f, 0.0)
    out_ref[...] = jnp.dot(f, kwf_ref[...],
                           preferred_element_type=jnp.float32) + bf_ref[...]


_EBLK = 6400


def kernel(x, edge_index, edge_attr, W1, b1, W2, b2, We, be, Wo, bo, Wf, bf):
    src = edge_index[0]
    dst = edge_index[1]
    pad = jnp.full((EPAD - E,), N, jnp.int32)
    src2d = jnp.concatenate([src, pad]).reshape(NT * NOP, BLKL)
    dst2d = jnp.concatenate([dst, pad]).reshape(NT * NOP, BLKL)
    x_pad = jnp.pad(x, ((0, NPAD - N), (0, 0)))
    ones2 = jnp.ones((BLKL, 16), jnp.float32)
    z8 = jnp.zeros((NSLICE, 8), jnp.float32)
    z16 = jnp.zeros((NSLICE, 16), jnp.float32)

    nblk = E // _EBLK
    ea_p = edge_attr.reshape(E // 16, 256)

    eye8 = jnp.eye(8, dtype=jnp.float32)
    eye16 = jnp.eye(16, dtype=jnp.float32)
    kw1 = jnp.kron(eye8, W1)
    kw2 = jnp.kron(eye16, W2)
    c16to8 = jnp.zeros((16, 8), jnp.float32).at[0].set(1.0)
    t8 = jnp.kron(eye16, c16to8)
    b1_256 = jnp.tile(b1, 16).reshape(1, 256)
    b2_128 = jnp.tile(b2, 16).reshape(1, 128)

    deg_p = _sc_degree(dst2d, ones2, z16)

    xws_p, dinv16_p = pl.pallas_call(
        _tc1_body,
        out_shape=(jax.ShapeDtypeStruct((1280, 128), jnp.float32),
                   jax.ShapeDtypeStruct((1280, 128), jnp.float32)),
    )(deg_p.reshape(2, 1280, 128), x_pad.reshape(1280, 1024), kw1)

    agg1_p = _sc_agg16(xws_p.reshape(NPAD, 16), src2d, dst2d, z16)

    h1ws_p = pl.pallas_call(
        _tc2_body,
        out_shape=jax.ShapeDtypeStruct((640, 128), jnp.float32),
    )(agg1_p.reshape(2, 640, 256), xws_p.reshape(640, 256),
      dinv16_p.reshape(640, 256), kw2, t8, b1_256)

    agg2_p = _sc_agg8(h1ws_p.reshape(NPAD, 8), src2d, dst2d, z8)

    h2_p = pl.pallas_call(
        _tc3_body,
        out_shape=jax.ShapeDtypeStruct((640, 128), jnp.float32),
    )(agg2_p.reshape(2, 640, 128), h1ws_p,
      dinv16_p.reshape(640, 256), t8, b2_128)

    h2s3, h2d3 = _sc_gather(h2_p.reshape(NPAD, 8), src2d, dst2d)
    h2s_p = h2s3.reshape(EPAD // 16, 128)
    h2d_p = h2d3.reshape(EPAD // 16, 128)

    eye16 = jnp.eye(16, dtype=jnp.float32)
    kwe = jnp.kron(eye16, We)
    kwo1 = jnp.kron(eye16, Wo[0:8])
    kwo2 = jnp.kron(eye16, Wo[8:16])
    kwo3 = jnp.kron(eye16, Wo[16:24])
    kwf = jnp.kron(eye16, Wf)
    be16 = jnp.tile(be, 16).reshape(1, 128)
    bo16 = jnp.tile(bo, 16).reshape(1, 128)
    bf16 = jnp.tile(bf, 16).reshape(1, 16)

    pblk = _EBLK // 16
    out_p = pl.pallas_call(
        _tc4_body,
        grid=(nblk,),
        in_specs=[
            pl.BlockSpec((pblk, 256), lambda i: (i, 0)),
            pl.BlockSpec((pblk, 128), lambda i: (i, 0)),
            pl.BlockSpec((pblk, 128), lambda i: (i, 0)),
            pl.BlockSpec((256, 128), lambda i: (0, 0)),
            pl.BlockSpec((1, 128), lambda i: (0, 0)),
            pl.BlockSpec((128, 128), lambda i: (0, 0)),
            pl.BlockSpec((1, 128), lambda i: (0, 0)),
            pl.BlockSpec((128, 128), lambda i: (0, 0)),
            pl.BlockSpec((128, 128), lambda i: (0, 0)),
            pl.BlockSpec((128, 16), lambda i: (0, 0)),
            pl.BlockSpec((1, 16), lambda i: (0, 0)),
        ],
        out_specs=pl.BlockSpec((pblk, 16), lambda i: (i, 0)),
        out_shape=jax.ShapeDtypeStruct((E // 16, 16), jnp.float32),
    )(ea_p, h2s_p, h2d_p, kwe, be16, kwo3, bo16, kwo1, kwo2, kwf, bf16)

    return out_p.reshape(E, 1)


# speedup vs baseline: 28.4156x; 1.1136x over previous
import functools

import jax
import jax.numpy as jnp
from jax import lax
from jax.experimental import pallas as pl
from jax.experimental.pallas import tpu as pltpu
from jax.experimental.pallas import tpu_sc as plsc

N = 10000
E = 320000
D_NODE = 128

NPAD = 10240
NT = 32
BLKL = 1024
NOP = 10
EPT = BLKL * NOP
EPAD = NT * EPT
NSLICE = NPAD // 16
RB = 4
LA = 2

_mesh = plsc.VectorSubcoreMesh(core_axis_name="c", subcore_axis_name="s")
_sc_params = pltpu.CompilerParams(use_tc_tiling_on_sc=False)
_sc_params_nl = pltpu.CompilerParams(use_tc_tiling_on_sc=False,
                                     needs_layout_passes=False)


def _wid(cid, sid):
    return sid * 2 + cid


@functools.partial(
    pl.kernel,
    out_type=jax.ShapeDtypeStruct((2, NPAD, 16), jnp.float32),
    mesh=_mesh,
    compiler_params=_sc_params,
    scratch_types=[
        pltpu.VMEM((NOP, BLKL), jnp.int32),
        pltpu.VMEM((BLKL, 16), jnp.float32),
        pltpu.VMEM_SHARED((NPAD, 16), jnp.float32),
        pltpu.SemaphoreType.DMA,
    ],
)
def _sc_degree(dst2d, ones2, zeros16, out, idx_v, ones_v, acc, sem):
    cid = lax.axis_index("c")
    sid = lax.axis_index("s")
    w = _wid(cid, sid)
    pltpu.sync_copy(zeros16, acc.at[pl.ds(sid * NSLICE, NSLICE)])
    pltpu.sync_copy(dst2d.at[pl.ds(w * NOP, NOP)], idx_v)
    pltpu.sync_copy(ones2, ones_v)
    plsc.subcore_barrier()
    descs = []
    for k in range(NOP):
        descs.append(pltpu.async_copy(
            ones_v, acc.at[idx_v.at[k]], sem, add=True))
    for d in descs:
        d.wait()
    plsc.subcore_barrier()
    pltpu.sync_copy(acc.at[pl.ds(sid * NSLICE, NSLICE)],
                    out.at[cid, pl.ds(sid * NSLICE, NSLICE)])


def _make_sc_agg(width):
    @functools.partial(
        pl.kernel,
        out_type=jax.ShapeDtypeStruct((2, NPAD, width), jnp.float32),
        mesh=_mesh,
        compiler_params=_sc_params,
        scratch_types=[
            pltpu.VMEM((NOP, BLKL), jnp.int32),
            pltpu.VMEM((NOP, BLKL), jnp.int32),
            [pltpu.VMEM((BLKL, width), jnp.float32)] * RB,
            [pltpu.SemaphoreType.DMA] * RB,
            [pltpu.SemaphoreType.DMA] * RB,
            pltpu.VMEM_SHARED((NPAD, width), jnp.float32),
        ],
    )
    def _sc_agg(table, src2d, dst2d, zeros, out,
                src_v, dst_v, bufs, gsems, wsems, acc):
        cid = lax.axis_index("c")
        sid = lax.axis_index("s")
        w = _wid(cid, sid)
        pltpu.sync_copy(zeros, acc.at[pl.ds(sid * NSLICE, NSLICE)])
        pltpu.sync_copy(src2d.at[pl.ds(w * NOP, NOP)], src_v)
        pltpu.sync_copy(dst2d.at[pl.ds(w * NOP, NOP)], dst_v)
        plsc.subcore_barrier()

        def gather(k):
            r = k % RB
            return pltpu.async_copy(table.at[src_v.at[k]], bufs[r], gsems[r])

        gd = {k: gather(k) for k in range(LA)}
        sd = {}
        drained = set()
        for k in range(NOP):
            r = k % RB
            gd[k].wait()
            sd[k] = pltpu.async_copy(
                bufs[r], acc.at[dst_v.at[k]], wsems[r], add=True)
            kn = k + LA
            if kn < NOP:
                prev = kn - RB
                if prev >= 0:
                    sd[prev].wait()
                    drained.add(prev)
                gd[kn] = gather(kn)
        for k in range(NOP):
            if k not in drained:
                sd[k].wait()
        plsc.subcore_barrier()
        pltpu.sync_copy(acc.at[pl.ds(sid * NSLICE, NSLICE)],
                        out.at[cid, pl.ds(sid * NSLICE, NSLICE)])

    return _sc_agg


_sc_agg16 = _make_sc_agg(16)
_sc_agg8 = _make_sc_agg(8)


@functools.partial(
    pl.kernel,
    out_type=(jax.ShapeDtypeStruct((NT * NOP, BLKL * 8), jnp.float32),
              jax.ShapeDtypeStruct((NT * NOP, BLKL * 8), jnp.float32)),
    mesh=_mesh,
    compiler_params=_sc_params_nl,
    scratch_types=[
        pltpu.VMEM((NPAD, 8), jnp.float32),
        pltpu.VMEM((BLKL,), jnp.int32),
        pltpu.VMEM((BLKL,), jnp.int32),
        [pltpu.VMEM((BLKL * 8,), jnp.float32)] * 2,
        [pltpu.VMEM((BLKL * 8,), jnp.float32)] * 2,
        [pltpu.SemaphoreType.DMA] * 2,
        [pltpu.SemaphoreType.DMA] * 2,
    ],
)
def _sc_gather(table, src2d, dst2d, out_s, out_d,
               h2t, idx_s, idx_d, bufs_s, bufs_d, wsems_s, wsems_d):
    cid = lax.axis_index("c")
    sid = lax.axis_index("s")
    w = _wid(cid, sid)
    pltpu.sync_copy(table, h2t)
    lane = lax.iota(jnp.int32, 16)
    c_div8 = lane >> 3
    c_mod8 = lane & 7

    def fill(idx_v, buf):
        @plsc.parallel_loop(0, BLKL // 2, unroll=8)
        def _body(i):
            node2 = plsc.load_gather(idx_v, [2 * i + c_div8])
            vals = plsc.load_gather(h2t, [node2, c_mod8])
            buf[pl.ds(i * 16, 16)] = vals

    wds, wdd = {}, {}
    for k in range(NOP):
        r = k % 2
        row = w * NOP + k
        if k >= 2:
            wds[k - 2].wait()
            wdd[k - 2].wait()
        pltpu.sync_copy(src2d.at[row], idx_s)
        pltpu.sync_copy(dst2d.at[row], idx_d)
        fill(idx_s, bufs_s[r])
        wds[k] = pltpu.async_copy(bufs_s[r], out_s.at[row], wsems_s[r])
        fill(idx_d, bufs_d[r])
        wdd[k] = pltpu.async_copy(bufs_d[r], out_d.at[row], wsems_d[r])
    wds[NOP - 2].wait()
    wdd[NOP - 2].wait()
    wds[NOP - 1].wait()
    wdd[NOP - 1].wait()


def _tc1_body(deg_ref, x_ref, kw1_ref, xws_ref, dinv_ref):
    dp = deg_ref[...]
    dinv16 = lax.rsqrt(dp[0] + dp[1] + 1.0)
    xw = jnp.dot(x_ref[...], kw1_ref[...],
                 preferred_element_type=jnp.float32)
    xws_ref[...] = xw * dinv16
    dinv_ref[...] = dinv16


def _tc2_body(agg_ref, xws_ref, dinv_ref, kw2_ref, t8_ref, b1_ref, out_ref):
    ap = agg_ref[...]
    dinv16 = dinv_ref[...]
    conv = (ap[0] + ap[1] + xws_ref[...]) * dinv16 + b1_ref[...]
    h1 = jnp.maximum(conv, 0.0)
    h1w = jnp.dot(h1, kw2_ref[...],
                  preferred_element_type=jnp.float32)
    dinv8 = jnp.dot(dinv16, t8_ref[...],
                    preferred_element_type=jnp.float32)
    out_ref[...] = h1w * dinv8


def _tc3_body(agg_ref, h1ws_ref, dinv_ref, t8_ref, b2_ref, out_ref):
    ap = agg_ref[...]
    dinv8 = jnp.dot(dinv_ref[...], t8_ref[...],
                    preferred_element_type=jnp.float32)
    conv = (ap[0] + ap[1] + h1ws_ref[...]) * dinv8 + b2_ref[...]
    out_ref[...] = jnp.maximum(conv, 0.0)


def _tc4_body(ea_ref, hs_ref, hd_ref, kwe_ref, be_ref, kwo3_ref, bo_ref,
              kwo1_ref, kwo2_ref, kwf_ref, bf_ref, out_ref):
    e8 = jnp.maximum(
        jnp.dot(ea_ref[...], kwe_ref[...], preferred_element_type=jnp.float32)
        + be_ref[...], 0.0)
    q = jnp.dot(e8, kwo3_ref[...],
                preferred_element_type=jnp.float32) + bo_ref[...]
    f = (jnp.dot(hs_ref[...], kwo1_ref[...],
                 preferred_element_type=jnp.float32)
         + jnp.dot(hd_ref[...], kwo2_ref[...],
                   preferred_element_type=jnp.float32)
         + q)
    f = jnp.maximum(f, 0.0)
    out_ref[...] = jnp.dot(f, kwf_ref[...],
                           preferred_element_type=jnp.float32) + bf_ref[...]


_EBLK = 6400


def kernel(x, edge_index, edge_attr, W1, b1, W2, b2, We, be, Wo, bo, Wf, bf):
    src = edge_index[0]
    dst = edge_index[1]
    pad = jnp.full((EPAD - E,), N, jnp.int32)
    src2d = jnp.concatenate([src, pad]).reshape(NT * NOP, BLKL)
    dst2d = jnp.concatenate([dst, pad]).reshape(NT * NOP, BLKL)
    x_pad = jnp.pad(x, ((0, NPAD - N), (0, 0)))
    ones2 = jnp.ones((BLKL, 16), jnp.float32)
    z8 = jnp.zeros((NSLICE, 8), jnp.float32)
    z16 = jnp.zeros((NSLICE, 16), jnp.float32)

    nblk = E // _EBLK
    ea_p = edge_attr.reshape(E // 16, 256)

    eye8 = jnp.eye(8, dtype=jnp.float32)
    eye16 = jnp.eye(16, dtype=jnp.float32)
    kw1 = jnp.kron(eye8, W1)
    kw2 = jnp.kron(eye16, W2)
    c16to8 = jnp.zeros((16, 8), jnp.float32).at[0].set(1.0)
    t8 = jnp.kron(eye16, c16to8)
    b1_256 = jnp.tile(b1, 16).reshape(1, 256)
    b2_128 = jnp.tile(b2, 16).reshape(1, 128)

    deg_p = _sc_degree(dst2d, ones2, z16)

    xws_p, dinv16_p = pl.pallas_call(
        _tc1_body,
        out_shape=(jax.ShapeDtypeStruct((1280, 128), jnp.float32),
                   jax.ShapeDtypeStruct((1280, 128), jnp.float32)),
    )(deg_p.reshape(2, 1280, 128), x_pad.reshape(1280, 1024), kw1)

    agg1_p = _sc_agg16(xws_p.reshape(NPAD, 16), src2d, dst2d, z16)

    h1ws_p = pl.pallas_call(
        _tc2_body,
        out_shape=jax.ShapeDtypeStruct((640, 128), jnp.float32),
    )(agg1_p.reshape(2, 640, 256), xws_p.reshape(640, 256),
      dinv16_p.reshape(640, 256), kw2, t8, b1_256)

    agg2_p = _sc_agg8(h1ws_p.reshape(NPAD, 8), src2d, dst2d, z8)

    h2_p = pl.pallas_call(
        _tc3_body,
        out_shape=jax.ShapeDtypeStruct((640, 128), jnp.float32),
    )(agg2_p.reshape(2, 640, 128), h1ws_p,
      dinv16_p.reshape(640, 256), t8, b2_128)

    h2s2, h2d2 = _sc_gather(h2_p.reshape(NPAD, 8), src2d, dst2d)
    h2s_p = h2s2.reshape(EPAD // 16, 128)
    h2d_p = h2d2.reshape(EPAD // 16, 128)

    eye16 = jnp.eye(16, dtype=jnp.float32)
    kwe = jnp.kron(eye16, We)
    kwo1 = jnp.kron(eye16, Wo[0:8])
    kwo2 = jnp.kron(eye16, Wo[8:16])
    kwo3 = jnp.kron(eye16, Wo[16:24])
    kwf = jnp.kron(eye16, Wf)
    be16 = jnp.tile(be, 16).reshape(1, 128)
    bo16 = jnp.tile(bo, 16).reshape(1, 128)
    bf16 = jnp.tile(bf, 16).reshape(1, 16)

    pblk = _EBLK // 16
    out_p = pl.pallas_call(
        _tc4_body,
        grid=(nblk,),
        in_specs=[
            pl.BlockSpec((pblk, 256), lambda i: (i, 0)),
            pl.BlockSpec((pblk, 128), lambda i: (i, 0)),
            pl.BlockSpec((pblk, 128), lambda i: (i, 0)),
            pl.BlockSpec((256, 128), lambda i: (0, 0)),
            pl.BlockSpec((1, 128), lambda i: (0, 0)),
            pl.BlockSpec((128, 128), lambda i: (0, 0)),
            pl.BlockSpec((1, 128), lambda i: (0, 0)),
            pl.BlockSpec((128, 128), lambda i: (0, 0)),
            pl.BlockSpec((128, 128), lambda i: (0, 0)),
            pl.BlockSpec((128, 16), lambda i: (0, 0)),
            pl.BlockSpec((1, 16), lambda i: (0, 0)),
        ],
        out_specs=pl.BlockSpec((pblk, 16), lambda i: (i, 0)),
        out_shape=jax.ShapeDtypeStruct((E // 16, 16), jnp.float32),
    )(ea_p, h2s_p, h2d_p, kwe, be16, kwo3, bo16, kwo1, kwo2, kwf, bf16)

    return out_p.reshape(E, 1)
